# Initial kernel scaffold; baseline (speedup 1.0000x reference)
import jax
import jax.numpy as jnp
from jax.experimental import pallas as pl


def kernel(data, param, wT, knots_x, knots_y, log_deriv):
    raise NotImplementedError("write your pallas kernel here")


# speedup vs baseline: 5674.5819x; 5674.5819x over previous
import functools

import jax
import jax.numpy as jnp
from jax import lax
from jax.experimental import pallas as pl
from jax.experimental.pallas import tpu as pltpu
from jax.experimental.pallas import tpu_sc as plsc

NDIM = 768
NCLASS = 16
NBIN = 100
NTOK = 4096
NBIN_PAD = 128
NROWS = NCLASS * NDIM

NWORKER = 32
DPW = NDIM // NWORKER
TCHUNK = 64
NCHUNK = NTOK // TCHUNK

_SORT_ROWS = 512
_MM_T = 512


def _sort_exp_body(kx_ref, ky_ref, ld_ref, sx_ref, sy_ref, dd_ref):
    ids = lax.broadcasted_iota(jnp.int32, (_SORT_ROWS, NBIN_PAD), 1)

    def bitonic(x):
        for k in (2, 4, 8, 16, 32, 64, 128):
            j = k // 2
            while j >= 1:
                pj = pltpu.roll(x, NBIN_PAD - j, axis=1)
                mj = pltpu.roll(x, j, axis=1)
                low = (ids & j) == 0
                partner = jnp.where(low, pj, mj)
                keep_min = low == ((ids & k) == 0)
                x = jnp.where(keep_min, jnp.minimum(x, partner),
                              jnp.maximum(x, partner))
                j //= 2
        return x

    sx_ref[...] = bitonic(kx_ref[...])[:, :NBIN]
    sy_ref[...] = bitonic(ky_ref[...])[:, :NBIN]
    dd_ref[...] = jnp.exp(ld_ref[...])[:, :NBIN]


def _mm_body(w_ref, a_ref, o_ref):
    o_ref[...] = lax.dot_general(
        w_ref[...], a_ref[...], (((0,), (1,)), ((), ())),
        preferred_element_type=jnp.float32)


def _final_body(data_ref, yt_ref, d0t_ref, dvt_ref, w_ref, o_ref, lj_ref):
    diff = yt_ref[...] - d0t_ref[...]
    o_ref[...] = data_ref[...] + lax.dot_general(
        diff, w_ref[...], (((0,), (1,)), ((), ())),
        preferred_element_type=jnp.float32)
    lj_ref[...] = jnp.sum(jnp.log(dvt_ref[...]), axis=0)[None, None, :]


def _spline_sc_body(d0_hbm, par_hbm, sx_hbm, sy_hbm, dd_hbm, y_hbm, dv_hbm,
                    sx_v, sy_v, dd_v, par_v, x_v, yo_v, dv_v):
    wid = lax.axis_index("s") * 2 + lax.axis_index("c")
    dbase = wid * DPW
    pltpu.sync_copy(par_hbm, par_v)
    pltpu.sync_copy(sx_hbm.at[:, pl.ds(dbase, DPW), :], sx_v)
    pltpu.sync_copy(sy_hbm.at[:, pl.ds(dbase, DPW), :], sy_v)
    pltpu.sync_copy(dd_hbm.at[:, pl.ds(dbase, DPW), :], dd_v)

    def chunk_body(ci, carry):
        t0 = ci * TCHUNK
        pltpu.sync_copy(d0_hbm.at[pl.ds(dbase, DPW), pl.ds(t0, TCHUNK)], x_v)

        def tv_body(tv, carry):
            toff = tv * 16
            c_vec = par_v[pl.ds(t0 + toff, 16)]

            def dl_body(dl, carry):
                dl_vec = jnp.broadcast_to(dl, (16,))
                v = x_v[dl, pl.ds(toff, 16)]
                idxp = jnp.zeros((16,), jnp.int32)
                for b in (64, 32, 16, 8, 4, 2, 1):
                    jj = idxp + b
                    jc = jnp.minimum(jj, NBIN)
                    xprobe = plsc.load_gather(sx_v, [c_vec, dl_vec, jc - 1])
                    take = (jj <= NBIN) & (xprobe < v)
                    idxp = jnp.where(take, jj, idxp)
                kk = jnp.clip(idxp - 1, 0, NBIN - 2)
                k1 = kk + 1
                xk = plsc.load_gather(sx_v, [c_vec, dl_vec, kk])
                xk1 = plsc.load_gather(sx_v, [c_vec, dl_vec, k1])
                yk = plsc.load_gather(sy_v, [c_vec, dl_vec, kk])
                yk1 = plsc.load_gather(sy_v, [c_vec, dl_vec, k1])
                dk = plsc.load_gather(dd_v, [c_vec, dl_vec, kk])
                dk1 = plsc.load_gather(dd_v, [c_vec, dl_vec, k1])
                w = xk1 - xk
                s = (yk1 - yk) / w
                xi = jnp.clip((v - xk) / w, 0.0, 1.0)
                omxi = 1.0 - xi
                xio = xi * omxi
                denom = s + (dk1 + dk - 2.0 * s) * xio
                y_sp = yk + (yk1 - yk) * (s * xi * xi + dk * xio) / denom
                deriv_sp = (s * s
                            * (dk1 * xi * xi + 2.0 * s * xio + dk * omxi * omxi)
                            / (denom * denom))
                below = (idxp == 0) & (v < xk)
                above = idxp >= NBIN
                y_out = jnp.where(below, yk + (v - xk) * dk,
                                  jnp.where(above, yk1 + (v - xk1) * dk1, y_sp))
                d_out = jnp.where(below, dk, jnp.where(above, dk1, deriv_sp))
                yo_v[dl, pl.ds(toff, 16)] = y_out
                dv_v[dl, pl.ds(toff, 16)] = d_out
                return carry

            return lax.fori_loop(0, DPW, dl_body, carry)

        lax.fori_loop(0, TCHUNK // 16, tv_body, 0)
        pltpu.sync_copy(yo_v, y_hbm.at[pl.ds(dbase, DPW), pl.ds(t0, TCHUNK)])
        pltpu.sync_copy(dv_v, dv_hbm.at[pl.ds(dbase, DPW), pl.ds(t0, TCHUNK)])
        return carry

    lax.fori_loop(0, NCHUNK, chunk_body, 0)


def kernel(data, param, wT, knots_x, knots_y, log_deriv):
    param32 = param.astype(jnp.int32)
    pad = ((0, 0), (0, NBIN_PAD - NBIN))
    kxp = jnp.pad(knots_x.reshape(NROWS, NBIN), pad, constant_values=jnp.inf)
    kyp = jnp.pad(knots_y.reshape(NROWS, NBIN), pad, constant_values=jnp.inf)
    ldp = jnp.pad(log_deriv.reshape(NROWS, NBIN), pad)

    iblk = pl.BlockSpec((_SORT_ROWS, NBIN_PAD), lambda i: (i, 0))
    oblk = pl.BlockSpec((_SORT_ROWS, NBIN), lambda i: (i, 0))
    sx, sy, dd = pl.pallas_call(
        _sort_exp_body,
        grid=(NROWS // _SORT_ROWS,),
        in_specs=[iblk, iblk, iblk],
        out_specs=[oblk, oblk, oblk],
        out_shape=[jax.ShapeDtypeStruct((NROWS, NBIN), jnp.float32)] * 3,
    )(kxp, kyp, ldp)
    sx = sx.reshape(NCLASS, NDIM, NBIN)
    sy = sy.reshape(NCLASS, NDIM, NBIN)
    dd = dd.reshape(NCLASS, NDIM, NBIN)

    data0t = pl.pallas_call(
        _mm_body,
        grid=(NTOK // _MM_T,),
        in_specs=[pl.BlockSpec((NDIM, NDIM), lambda i: (0, 0)),
                  pl.BlockSpec((_MM_T, NDIM), lambda i: (i, 0))],
        out_specs=pl.BlockSpec((NDIM, _MM_T), lambda i: (0, i)),
        out_shape=jax.ShapeDtypeStruct((NDIM, NTOK), jnp.float32),
    )(wT, data)

    spline = pl.kernel(
        _spline_sc_body,
        out_type=[jax.ShapeDtypeStruct((NDIM, NTOK), jnp.float32),
                  jax.ShapeDtypeStruct((NDIM, NTOK), jnp.float32)],
        mesh=plsc.VectorSubcoreMesh(core_axis_name="c", subcore_axis_name="s"),
        compiler_params=pltpu.CompilerParams(use_tc_tiling_on_sc=False,
                                             needs_layout_passes=False),
        scratch_types=[
            pltpu.VMEM((NCLASS, DPW, NBIN), jnp.float32),
            pltpu.VMEM((NCLASS, DPW, NBIN), jnp.float32),
            pltpu.VMEM((NCLASS, DPW, NBIN), jnp.float32),
            pltpu.VMEM((NTOK,), jnp.int32),
            pltpu.VMEM((DPW, TCHUNK), jnp.float32),
            pltpu.VMEM((DPW, TCHUNK), jnp.float32),
            pltpu.VMEM((DPW, TCHUNK), jnp.float32),
        ],
    )
    yt, dvt = spline(data0t, param32, sx, sy, dd)

    data_out, lj = pl.pallas_call(
        _final_body,
        grid=(NTOK // _MM_T,),
        in_specs=[pl.BlockSpec((_MM_T, NDIM), lambda i: (i, 0)),
                  pl.BlockSpec((NDIM, _MM_T), lambda i: (0, i)),
                  pl.BlockSpec((NDIM, _MM_T), lambda i: (0, i)),
                  pl.BlockSpec((NDIM, _MM_T), lambda i: (0, i)),
                  pl.BlockSpec((NDIM, NDIM), lambda i: (0, 0))],
        out_specs=[pl.BlockSpec((_MM_T, NDIM), lambda i: (i, 0)),
                   pl.BlockSpec((1, 1, _MM_T), lambda i: (i, 0, 0))],
        out_shape=[jax.ShapeDtypeStruct((NTOK, NDIM), jnp.float32),
                   jax.ShapeDtypeStruct((NTOK // _MM_T, 1, _MM_T), jnp.float32)],
    )(data, yt, data0t, dvt, wT)
    return data_out, lj.reshape(NTOK)


# speedup vs baseline: 9841.8792x; 1.7344x over previous
import functools

import jax
import jax.numpy as jnp
from jax import lax
from jax.experimental import pallas as pl
from jax.experimental.pallas import tpu as pltpu
from jax.experimental.pallas import tpu_sc as plsc

NDIM = 768
NCLASS = 16
NBIN = 100
NTOK = 4096
NBIN_PAD = 128
NROWS = NCLASS * NDIM

NWORKER = 32
DPW = NDIM // NWORKER
TCHUNK = 64
NCHUNK = NTOK // TCHUNK

_SORT_ROWS = 512
_MM_T = 512


def _sort_exp_body(kx_ref, ky_ref, ld_ref, sx_ref, sy_ref, dd_ref):
    ids = lax.broadcasted_iota(jnp.int32, (_SORT_ROWS, NBIN_PAD), 1)

    def bitonic(x):
        for k in (2, 4, 8, 16, 32, 64, 128):
            j = k // 2
            while j >= 1:
                pj = pltpu.roll(x, NBIN_PAD - j, axis=1)
                mj = pltpu.roll(x, j, axis=1)
                low = (ids & j) == 0
                partner = jnp.where(low, pj, mj)
                keep_min = low == ((ids & k) == 0)
                x = jnp.where(keep_min, jnp.minimum(x, partner),
                              jnp.maximum(x, partner))
                j //= 2
        return x

    sx_ref[...] = bitonic(kx_ref[...])[:, :NBIN]
    sy_ref[...] = bitonic(ky_ref[...])[:, :NBIN]
    dd_ref[...] = jnp.exp(ld_ref[...])[:, :NBIN]


def _mm_body(w_ref, a_ref, o_ref):
    o_ref[...] = lax.dot_general(
        w_ref[...], a_ref[...], (((0,), (1,)), ((), ())),
        preferred_element_type=jnp.float32)


def _final_body(data_ref, yt_ref, d0t_ref, dvt_ref, w_ref, o_ref, lj_ref):
    diff = yt_ref[...] - d0t_ref[...]
    o_ref[...] = data_ref[...] + lax.dot_general(
        diff, w_ref[...], (((0,), (1,)), ((), ())),
        preferred_element_type=jnp.float32)
    lj_ref[...] = jnp.sum(jnp.log(dvt_ref[...]), axis=0)[None, None, :]


def _spline_sc_body(d0_hbm, par_hbm, sx_hbm, sy_hbm, dd_hbm, y_hbm, dv_hbm,
                    sx_v, sy_v, dd_v, par_v, x_v, yo_v, dv_v):
    wid = lax.axis_index("s") * 2 + lax.axis_index("c")
    dbase = wid * DPW
    pltpu.sync_copy(par_hbm, par_v)
    pltpu.sync_copy(sx_hbm.at[wid], sx_v)
    pltpu.sync_copy(sy_hbm.at[wid], sy_v)
    pltpu.sync_copy(dd_hbm.at[wid], dd_v)
    ngrp = (TCHUNK // 16) * DPW

    def chunk_body(ci, carry):
        t0 = ci * TCHUNK
        pltpu.sync_copy(d0_hbm.at[pl.ds(dbase, DPW), pl.ds(t0, TCHUNK)], x_v)

        @plsc.parallel_loop(0, ngrp, unroll=4)
        def grp_body(g):
            tv = lax.rem(g, TCHUNK // 16)
            dl = lax.div(g, TCHUNK // 16)
            toff = tv * 16
            c_vec = par_v[pl.ds(t0 + toff, 16)]
            base = c_vec * (DPW * NBIN) + dl * NBIN
            v = x_v[dl, pl.ds(toff, 16)]
            idxp = jnp.zeros((16,), jnp.int32)
            basem1 = base - 1
            for b in (64, 32, 16, 8, 4, 2, 1):
                jj = idxp + b
                jc = jnp.minimum(jj, NBIN)
                xprobe = plsc.load_gather(sx_v, [basem1 + jc])
                take = (jj <= NBIN) & (xprobe < v)
                idxp = jnp.where(take, jj, idxp)
            kk = jnp.clip(idxp - 1, 0, NBIN - 2)
            bk = base + kk
            bk1 = bk + 1
            xk = plsc.load_gather(sx_v, [bk])
            xk1 = plsc.load_gather(sx_v, [bk1])
            yk = plsc.load_gather(sy_v, [bk])
            yk1 = plsc.load_gather(sy_v, [bk1])
            dk = plsc.load_gather(dd_v, [bk])
            dk1 = plsc.load_gather(dd_v, [bk1])
            w = xk1 - xk
            s = (yk1 - yk) / w
            xi = jnp.clip((v - xk) / w, 0.0, 1.0)
            omxi = 1.0 - xi
            xio = xi * omxi
            denom = s + (dk1 + dk - 2.0 * s) * xio
            y_sp = yk + (yk1 - yk) * (s * xi * xi + dk * xio) / denom
            deriv_sp = (s * s
                        * (dk1 * xi * xi + 2.0 * s * xio + dk * omxi * omxi)
                        / (denom * denom))
            below = (idxp == 0) & (v < xk)
            above = idxp >= NBIN
            y_out = jnp.where(below, yk + (v - xk) * dk,
                              jnp.where(above, yk1 + (v - xk1) * dk1, y_sp))
            d_out = jnp.where(below, dk, jnp.where(above, dk1, deriv_sp))
            yo_v[dl, pl.ds(toff, 16)] = y_out
            dv_v[dl, pl.ds(toff, 16)] = d_out

        pltpu.sync_copy(yo_v, y_hbm.at[pl.ds(dbase, DPW), pl.ds(t0, TCHUNK)])
        pltpu.sync_copy(dv_v, dv_hbm.at[pl.ds(dbase, DPW), pl.ds(t0, TCHUNK)])
        return carry

    lax.fori_loop(0, NCHUNK, chunk_body, 0)


def kernel(data, param, wT, knots_x, knots_y, log_deriv):
    param32 = param.astype(jnp.int32)
    pad = ((0, 0), (0, NBIN_PAD - NBIN))
    kxp = jnp.pad(knots_x.reshape(NROWS, NBIN), pad, constant_values=jnp.inf)
    kyp = jnp.pad(knots_y.reshape(NROWS, NBIN), pad, constant_values=jnp.inf)
    ldp = jnp.pad(log_deriv.reshape(NROWS, NBIN), pad)

    iblk = pl.BlockSpec((_SORT_ROWS, NBIN_PAD), lambda i: (i, 0))
    oblk = pl.BlockSpec((_SORT_ROWS, NBIN), lambda i: (i, 0))
    sx, sy, dd = pl.pallas_call(
        _sort_exp_body,
        grid=(NROWS // _SORT_ROWS,),
        in_specs=[iblk, iblk, iblk],
        out_specs=[oblk, oblk, oblk],
        out_shape=[jax.ShapeDtypeStruct((NROWS, NBIN), jnp.float32)] * 3,
    )(kxp, kyp, ldp)
    def _worker_major(t):
        t = t.reshape(NCLASS, NWORKER, DPW, NBIN).transpose(1, 0, 2, 3)
        return t.reshape(NWORKER, NCLASS * DPW * NBIN)

    sx = _worker_major(sx)
    sy = _worker_major(sy)
    dd = _worker_major(dd)

    data0t = pl.pallas_call(
        _mm_body,
        grid=(NTOK // _MM_T,),
        in_specs=[pl.BlockSpec((NDIM, NDIM), lambda i: (0, 0)),
                  pl.BlockSpec((_MM_T, NDIM), lambda i: (i, 0))],
        out_specs=pl.BlockSpec((NDIM, _MM_T), lambda i: (0, i)),
        out_shape=jax.ShapeDtypeStruct((NDIM, NTOK), jnp.float32),
    )(wT, data)

    spline = pl.kernel(
        _spline_sc_body,
        out_type=[jax.ShapeDtypeStruct((NDIM, NTOK), jnp.float32),
                  jax.ShapeDtypeStruct((NDIM, NTOK), jnp.float32)],
        mesh=plsc.VectorSubcoreMesh(core_axis_name="c", subcore_axis_name="s"),
        compiler_params=pltpu.CompilerParams(use_tc_tiling_on_sc=False,
                                             needs_layout_passes=False),
        scratch_types=[
            pltpu.VMEM((NCLASS * DPW * NBIN,), jnp.float32),
            pltpu.VMEM((NCLASS * DPW * NBIN,), jnp.float32),
            pltpu.VMEM((NCLASS * DPW * NBIN,), jnp.float32),
            pltpu.VMEM((NTOK,), jnp.int32),
            pltpu.VMEM((DPW, TCHUNK), jnp.float32),
            pltpu.VMEM((DPW, TCHUNK), jnp.float32),
            pltpu.VMEM((DPW, TCHUNK), jnp.float32),
        ],
    )
    yt, dvt = spline(data0t, param32, sx, sy, dd)

    data_out, lj = pl.pallas_call(
        _final_body,
        grid=(NTOK // _MM_T,),
        in_specs=[pl.BlockSpec((_MM_T, NDIM), lambda i: (i, 0)),
                  pl.BlockSpec((NDIM, _MM_T), lambda i: (0, i)),
                  pl.BlockSpec((NDIM, _MM_T), lambda i: (0, i)),
                  pl.BlockSpec((NDIM, _MM_T), lambda i: (0, i)),
                  pl.BlockSpec((NDIM, NDIM), lambda i: (0, 0))],
        out_specs=[pl.BlockSpec((_MM_T, NDIM), lambda i: (i, 0)),
                   pl.BlockSpec((1, 1, _MM_T), lambda i: (i, 0, 0))],
        out_shape=[jax.ShapeDtypeStruct((NTOK, NDIM), jnp.float32),
                   jax.ShapeDtypeStruct((NTOK // _MM_T, 1, _MM_T), jnp.float32)],
    )(data, yt, data0t, dvt, wT)
    return data_out, lj.reshape(NTOK)
